# Initial kernel scaffold; baseline (speedup 1.0000x reference)
import jax
import jax.numpy as jnp
from jax.experimental import pallas as pl


def kernel(batch, enc_w1, enc_b1, enc_w2, enc_b2, enc_w3, enc_b3, g1_wl, g1_bl, g1_wr, g1_br, g1_att, g1_bias, g2_wl, g2_bl, g2_wr, g2_br, g2_att, g2_bias, g3_wl, g3_bl, g3_wr, g3_br, g3_att, g3_bias, g4_wl, g4_bl, g4_wr, g4_br, g4_att, g4_bias, label_w, label_b, value_w, value_b, skip_w, skip_b):
    raise NotImplementedError("write your pallas kernel here")


# speedup vs baseline: 38.4225x; 38.4225x over previous
import functools

import jax
import jax.numpy as jnp
from jax.experimental import pallas as pl
from jax.experimental.pallas import tpu as pltpu

B, N, IN_DIM, OUT_DIM, HID, K = 64, 256, 5, 3, 64, 8
ALPHA = 0.1
_INTERPRET = False

_W_ENC2, _W_ENC3 = 0, 1
_W_G2L, _W_G2R, _W_G3L, _W_G3R, _W_G4L, _W_G4R = 2, 3, 4, 5, 6, 7
_W_LAB, _W_VAL, _W_SKIP = 8, 9, 10

_B_ENC1, _B_ENC2, _B_ENC3 = 0, 1, 2
_B_G = [(3 + 4 * i, 4 + 4 * i, 5 + 4 * i, 6 + 4 * i) for i in range(4)]
_B_LAB, _B_VAL, _B_SKIP = 19, 20, 21
_B_G1WL, _B_G1WR = 22, 23


def _rt(x):
    return x.astype(jnp.bfloat16).astype(jnp.float32)


def _mm_bb(a, b):
    return jnp.dot(a.astype(jnp.bfloat16), b.astype(jnp.bfloat16),
                   preferred_element_type=jnp.float32)


def _mm_xw(x, w):
    return jnp.dot(_rt(x), w, preferred_element_type=jnp.float32,
                   precision=jax.lax.Precision.HIGHEST)


def _mm_ff(a, b):
    return jnp.dot(a, b, preferred_element_type=jnp.float32,
                   precision=jax.lax.Precision.HIGHEST)


def _gat_layer(xl, xr, oh, bref, b_idx):
    _, _, att, bias = b_idx
    att_row = bref[att:att + 1, :]
    g = []
    e = []
    for k in range(K):
        gk = jnp.dot(oh[k], xl, preferred_element_type=jnp.float32,
                     precision=jax.lax.Precision.HIGHEST)
        g.append(gk)
        mk = gk + xr
        mk = jnp.where(mk >= 0, mk, 0.2 * mk)
        e.append(jnp.sum(_rt(mk) * att_row, axis=1, keepdims=True))
    e = jnp.concatenate(e, axis=1)
    emax = jnp.max(e, axis=1, keepdims=True)
    ex = jnp.exp(e - emax)
    denom = jnp.sum(ex, axis=1, keepdims=True)
    alpha = ex / denom
    out = bref[bias:bias + 1, :] + jnp.zeros_like(xl)
    for k in range(K):
        out = out + alpha[:, k:k + 1] * g[k]
    return out


def _obs_kernel(batch_ref, w1_ref, w_ref, b_ref, out_ref):
    obs = batch_ref[0]
    h = _mm_bb(obs, w1_ref[...]) + b_ref[_B_ENC1:_B_ENC1 + 1, :]
    h = jnp.maximum(h, 0.0)
    h = _mm_bb(h, w_ref[_W_ENC2]) + b_ref[_B_ENC2:_B_ENC2 + 1, :]
    h = jnp.maximum(h, 0.0)
    latent = _mm_bb(h, w_ref[_W_ENC3]) + b_ref[_B_ENC3:_B_ENC3 + 1, :]

    px = latent[:, 0:1]
    py = latent[:, 1:2]
    row_i = jax.lax.broadcasted_iota(jnp.int32, (N, N), 0)
    col_i = jax.lax.broadcasted_iota(jnp.int32, (N, N), 1)
    col_f = col_i.astype(jnp.float32)
    diag = row_i == col_i
    xj = jnp.max(jnp.where(diag, px, -3.0e38), axis=0, keepdims=True)
    yj = jnp.max(jnp.where(diag, py, -3.0e38), axis=0, keepdims=True)
    dx = px - xj
    dy = py - yj
    d2 = dx * dx + dy * dy + jnp.where(diag, 1e9, 0.0)

    oh = []
    idx_cols = []
    dist_cols = []
    d2_work = d2
    for k in range(K):
        mk = jnp.min(d2_work, axis=1, keepdims=True)
        cand = jnp.where(d2_work == mk, col_f, 3.0e38)
        idx_f = jnp.min(cand, axis=1, keepdims=True)
        sel = col_f == idx_f
        oh.append(sel.astype(jnp.float32))
        idx_cols.append(idx_f)
        dist_cols.append(mk)
        d2_work = jnp.where(sel, 3.0e38, d2_work)
    idx_mat = jnp.concatenate(idx_cols, axis=1)
    dist_mat = jnp.concatenate(dist_cols, axis=1)
    eattr = jnp.sqrt(jnp.maximum(dist_mat, 1e-12))

    x0 = latent[:, 2:3]
    xl1 = x0 * b_ref[_B_G1WL:_B_G1WL + 1, :] + b_ref[_B_G[0][0]:_B_G[0][0] + 1, :]
    xr1 = x0 * b_ref[_B_G1WR:_B_G1WR + 1, :] + b_ref[_B_G[0][1]:_B_G[0][1] + 1, :]
    x1 = jnp.maximum(_gat_layer(xl1, xr1, oh, b_ref, _B_G[0]), 0.0)

    def _proj(x, wl_i, wr_i, b_idx):
        xl = _mm_xw(x, w_ref[wl_i]) + b_ref[b_idx[0]:b_idx[0] + 1, :]
        xr = _mm_xw(x, w_ref[wr_i]) + b_ref[b_idx[1]:b_idx[1] + 1, :]
        return xl, xr

    x2 = jnp.maximum(_gat_layer(*_proj(x1, _W_G2L, _W_G2R, _B_G[1]), oh, b_ref, _B_G[1]), 0.0)
    skip = _mm_ff(latent, w_ref[_W_SKIP]) + b_ref[_B_SKIP:_B_SKIP + 1, :]
    x3 = jnp.maximum(_gat_layer(*_proj(x2, _W_G3L, _W_G3R, _B_G[2]), oh, b_ref, _B_G[2]) + ALPHA * skip, 0.0)
    x4 = jnp.maximum(_gat_layer(*_proj(x2, _W_G4L, _W_G4R, _B_G[3]), oh, b_ref, _B_G[3]) + ALPHA * skip, 0.0)
    logits = _mm_xw(x3, w_ref[_W_LAB]) + b_ref[_B_LAB:_B_LAB + 1, :]
    values = _mm_xw(x4, w_ref[_W_VAL]) + b_ref[_B_VAL:_B_VAL + 1, :]

    out_tile = jnp.concatenate([
        logits[:, 0:4],
        values[:, 0:1],
        latent[:, 0:3],
        eattr,
        idx_mat,
        jnp.zeros((N, 104), jnp.float32),
    ], axis=1)
    out_ref[0] = out_tile


def _pad_w(w, rows=HID, row_off=0):
    out = jnp.zeros((rows, HID), jnp.float32)
    return out.at[row_off:row_off + w.shape[0], :w.shape[1]].set(w)


def _pad_b(b):
    return jnp.zeros((HID,), jnp.float32).at[:b.shape[0]].set(b)


@jax.jit
def kernel(batch, enc_w1, enc_b1, enc_w2, enc_b2, enc_w3, enc_b3,
           g1_wl, g1_bl, g1_wr, g1_br, g1_att, g1_bias,
           g2_wl, g2_bl, g2_wr, g2_br, g2_att, g2_bias,
           g3_wl, g3_bl, g3_wr, g3_br, g3_att, g3_bias,
           g4_wl, g4_bl, g4_wr, g4_br, g4_att, g4_bias,
           label_w, label_b, value_w, value_b, skip_w, skip_b):
    batchp = jnp.pad(batch, ((0, 0), (0, 0), (0, 8 - IN_DIM)))
    w1p = _pad_w(enc_w1, rows=8)
    wstack = jnp.stack([
        enc_w2,
        _pad_w(enc_w3),
        g2_wl, g2_wr, g3_wl, g3_wr, g4_wl, g4_wr,
        _pad_w(label_w),
        _pad_w(value_w),
        _pad_w(skip_w),
    ])
    brows = [enc_b1, enc_b2, _pad_b(enc_b3)]
    for bl, br, att, bias in [(g1_bl, g1_br, g1_att, g1_bias),
                              (g2_bl, g2_br, g2_att, g2_bias),
                              (g3_bl, g3_br, g3_att, g3_bias),
                              (g4_bl, g4_br, g4_att, g4_bias)]:
        brows += [bl, br, att, bias]
    brows += [_pad_b(label_b), _pad_b(value_b), skip_b,
              g1_wl[0], g1_wr[0]]
    bstack = jnp.stack(brows)

    out = pl.pallas_call(
        _obs_kernel,
        grid=(B,),
        in_specs=[
            pl.BlockSpec((1, N, 8), lambda b: (b, 0, 0)),
            pl.BlockSpec((8, HID), lambda b: (0, 0)),
            pl.BlockSpec((11, HID, HID), lambda b: (0, 0, 0)),
            pl.BlockSpec((24, HID), lambda b: (0, 0)),
        ],
        out_specs=pl.BlockSpec((1, N, 128), lambda b: (b, 0, 0)),
        out_shape=jax.ShapeDtypeStruct((B, N, 128), jnp.float32),
        compiler_params=pltpu.CompilerParams(
            dimension_semantics=("arbitrary",),
        ),
        interpret=_INTERPRET,
    )(batchp, w1p, wstack, bstack)

    logits = out[:, :, 0:4]
    values = out[:, :, 4:5]
    latents = out[:, :, 5:8]
    eattr = out[:, :, 8:16].reshape(B, N * K)
    src = out[:, :, 16:24].astype(jnp.int32).reshape(B, N * K)
    dst = jnp.broadcast_to(jnp.repeat(jnp.arange(N, dtype=jnp.int32), K)[None, :], (B, N * K))
    eidx = jnp.stack([src, dst], axis=1)
    return (batch[:, :, :4], batch[:, :, 4:5], logits, values, latents, eidx, eattr)


# speedup vs baseline: 54.8540x; 1.4277x over previous
import jax
import jax.numpy as jnp
from jax.experimental import pallas as pl
from jax.experimental.pallas import tpu as pltpu

B, N, IN_DIM, OUT_DIM, HID, K = 64, 256, 5, 3, 64, 8
ALPHA = 0.1
_INTERPRET = False

_W_ENC2, _W_ENC3 = 0, 1
_W_G2L, _W_G2R, _W_G3L, _W_G3R, _W_G4L, _W_G4R = 2, 3, 4, 5, 6, 7

_B_ENC1, _B_ENC2, _B_ENC3 = 0, 1, 2
_B_G = [(3 + 4 * i, 4 + 4 * i, 5 + 4 * i, 6 + 4 * i) for i in range(4)]
_B_LAB, _B_VAL, _B_SKIP = 19, 20, 21
_B_G1WL, _B_G1WR = 22, 23
_B_LABW = 24
_B_VALW = 28


def _bf(x):
    return x.astype(jnp.bfloat16)


def _rt(x):
    return x.astype(jnp.bfloat16).astype(jnp.float32)


def _dot(a, b):
    return jnp.dot(a, b, preferred_element_type=jnp.float32)


def _split3(x):
    p0 = _bf(x)
    r1 = x - p0.astype(jnp.float32)
    p1 = _bf(r1)
    p2 = _bf(r1 - p1.astype(jnp.float32))
    return p0, p1, p2


def _mm_xw(x, w_hi):
    return _dot(_bf(x), w_hi)


def _gather(oh, x):
    p0, p1, p2 = _split3(x)
    return [(_dot(oh_k, p0) + _dot(oh_k, p1)) + _dot(oh_k, p2) for oh_k in oh]


def _attend(g, xr, bref, b_idx):
    _, _, att, bias = b_idx
    att_row = bref[att:att + 1, :]
    e = []
    for k in range(K):
        mk = g[k] + xr
        mk = jnp.where(mk >= 0, mk, 0.2 * mk)
        e.append(jnp.sum(_rt(mk) * _rt(att_row), axis=1, keepdims=True))
    e = jnp.concatenate(e, axis=1)
    emax = jnp.max(e, axis=1, keepdims=True)
    ex = jnp.exp(e - emax)
    denom = jnp.sum(ex, axis=1, keepdims=True)
    alpha = ex / denom
    out = bref[bias:bias + 1, :] + jnp.zeros_like(xr)
    for k in range(K):
        out = out + alpha[:, k:k + 1] * g[k]
    return out


def _obs_kernel(batch_ref, w1_ref, whi_ref, skipw_ref, b_ref, out_ref):
    obs = batch_ref[0]
    h = _dot(_bf(obs), w1_ref[...]) + b_ref[_B_ENC1:_B_ENC1 + 1, :]
    h = jnp.maximum(h, 0.0)
    h = _dot(_bf(h), whi_ref[_W_ENC2]) + b_ref[_B_ENC2:_B_ENC2 + 1, :]
    h = jnp.maximum(h, 0.0)
    latent = _dot(_bf(h), whi_ref[_W_ENC3]) + b_ref[_B_ENC3:_B_ENC3 + 1, :]

    px = latent[:, 0:1]
    py = latent[:, 1:2]
    row_i = jax.lax.broadcasted_iota(jnp.int32, (N, N), 0)
    col_i = jax.lax.broadcasted_iota(jnp.int32, (N, N), 1)
    col_f = col_i.astype(jnp.float32)
    diag = row_i == col_i
    xj = jnp.max(jnp.where(diag, px, -3.0e38), axis=0, keepdims=True)
    yj = jnp.max(jnp.where(diag, py, -3.0e38), axis=0, keepdims=True)
    dx = px - xj
    dy = py - yj
    d2 = dx * dx + dy * dy + jnp.where(diag, 1e9, 0.0)

    oh = []
    sel_masks = []
    idx_cols = []
    dist_cols = []
    d2_work = d2
    for k in range(K):
        mk = jnp.min(d2_work, axis=1, keepdims=True)
        cand = jnp.where(d2_work == mk, col_f, 3.0e38)
        idx_f = jnp.min(cand, axis=1, keepdims=True)
        sel = col_f == idx_f
        sel_masks.append(sel)
        oh.append(sel.astype(jnp.bfloat16))
        idx_cols.append(idx_f)
        dist_cols.append(mk)
        d2_work = jnp.where(sel, 3.0e38, d2_work)
    idx_mat = jnp.concatenate(idx_cols, axis=1)
    dist_mat = jnp.concatenate(dist_cols, axis=1)
    eattr = jnp.sqrt(jnp.maximum(dist_mat, 1e-12))

    x0 = latent[:, 2:3]
    x0_row = jnp.max(jnp.where(diag, x0, -3.0e38), axis=0, keepdims=True)
    wl1 = b_ref[_B_G1WL:_B_G1WL + 1, :]
    bl1 = b_ref[_B_G[0][0]:_B_G[0][0] + 1, :]
    br1 = b_ref[_B_G[0][1]:_B_G[0][1] + 1, :]
    g1 = []
    for k in range(K):
        gx0 = jnp.sum(jnp.where(sel_masks[k], x0_row, 0.0), axis=1, keepdims=True)
        g1.append(gx0 * wl1 + bl1)
    xr1 = x0 * b_ref[_B_G1WR:_B_G1WR + 1, :] + br1
    x1 = jnp.maximum(_attend(g1, xr1, b_ref, _B_G[0]), 0.0)

    xl2 = _mm_xw(x1, whi_ref[_W_G2L]) + b_ref[_B_G[1][0]:_B_G[1][0] + 1, :]
    xr2 = _mm_xw(x1, whi_ref[_W_G2R]) + b_ref[_B_G[1][1]:_B_G[1][1] + 1, :]
    g2 = _gather(oh, xl2)
    x2 = jnp.maximum(_attend(g2, xr2, b_ref, _B_G[1]), 0.0)

    gx2 = _gather(oh, x2)
    skip = _dot(_bf(latent), skipw_ref[...]) + b_ref[_B_SKIP:_B_SKIP + 1, :]
    sk = ALPHA * skip

    def _branch(wl_i, wr_i, b_idx):
        bl = b_ref[b_idx[0]:b_idx[0] + 1, :]
        g = [_mm_xw(gx2_k, whi_ref[wl_i]) + bl for gx2_k in gx2]
        xr = _mm_xw(x2, whi_ref[wr_i]) + b_ref[b_idx[1]:b_idx[1] + 1, :]
        return jnp.maximum(_attend(g, xr, b_ref, b_idx) + sk, 0.0)

    x3 = _branch(_W_G3L, _W_G3R, _B_G[2])
    x4 = _branch(_W_G4L, _W_G4R, _B_G[3])
    lab_cols = [jnp.sum(_rt(x3) * _rt(b_ref[_B_LABW + c:_B_LABW + c + 1, :]),
                        axis=1, keepdims=True)
                + b_ref[_B_LAB:_B_LAB + 1, c:c + 1] for c in range(4)]
    logits = jnp.concatenate(lab_cols, axis=1)
    values = (jnp.sum(_rt(x4) * _rt(b_ref[_B_VALW:_B_VALW + 1, :]), axis=1, keepdims=True)
              + b_ref[_B_VAL:_B_VAL + 1, 0:1])

    out_tile = jnp.concatenate([
        logits,
        values,
        latent[:, 0:3],
        eattr,
        idx_mat,
        jnp.zeros((N, 104), jnp.float32),
    ], axis=1)
    out_ref[0] = out_tile


def _pad_w(w, rows=HID):
    out = jnp.zeros((rows, HID), jnp.float32)
    return out.at[:w.shape[0], :w.shape[1]].set(w)


def _pad_b(b):
    return jnp.zeros((HID,), jnp.float32).at[:b.shape[0]].set(b)


@jax.jit
def kernel(batch, enc_w1, enc_b1, enc_w2, enc_b2, enc_w3, enc_b3,
           g1_wl, g1_bl, g1_wr, g1_br, g1_att, g1_bias,
           g2_wl, g2_bl, g2_wr, g2_br, g2_att, g2_bias,
           g3_wl, g3_bl, g3_wr, g3_br, g3_att, g3_bias,
           g4_wl, g4_bl, g4_wr, g4_br, g4_att, g4_bias,
           label_w, label_b, value_w, value_b, skip_w, skip_b):
    batchp = jnp.pad(batch, ((0, 0), (0, 0), (0, 8 - IN_DIM)))
    w1p = _pad_w(enc_w1, rows=8).astype(jnp.bfloat16)
    wstack = jnp.stack([
        enc_w2,
        _pad_w(enc_w3),
        g2_wl, g2_wr, g3_wl, g3_wr, g4_wl, g4_wr,
    ])
    whi = wstack.astype(jnp.bfloat16)
    skip_wp = _pad_w(skip_w).astype(jnp.bfloat16)
    brows = [enc_b1, enc_b2, _pad_b(enc_b3)]
    for bl, br, att, bias in [(g1_bl, g1_br, g1_att, g1_bias),
                              (g2_bl, g2_br, g2_att, g2_bias),
                              (g3_bl, g3_br, g3_att, g3_bias),
                              (g4_bl, g4_br, g4_att, g4_bias)]:
        brows += [bl, br, att, bias]
    brows += [_pad_b(label_b), _pad_b(value_b), skip_b,
              g1_wl[0], g1_wr[0]]
    brows += [label_w[:, c] for c in range(4)]
    brows += [value_w[:, 0],
              jnp.zeros((HID,), jnp.float32), jnp.zeros((HID,), jnp.float32),
              jnp.zeros((HID,), jnp.float32)]
    bstack = jnp.stack(brows)

    out = pl.pallas_call(
        _obs_kernel,
        grid=(B,),
        in_specs=[
            pl.BlockSpec((1, N, 8), lambda b: (b, 0, 0)),
            pl.BlockSpec((8, HID), lambda b: (0, 0)),
            pl.BlockSpec((8, HID, HID), lambda b: (0, 0, 0)),
            pl.BlockSpec((HID, HID), lambda b: (0, 0)),
            pl.BlockSpec((32, HID), lambda b: (0, 0)),
        ],
        out_specs=pl.BlockSpec((1, N, 128), lambda b: (b, 0, 0)),
        out_shape=jax.ShapeDtypeStruct((B, N, 128), jnp.float32),
        compiler_params=pltpu.CompilerParams(
            dimension_semantics=("arbitrary",),
        ),
        interpret=_INTERPRET,
    )(batchp, w1p, whi, skip_wp, bstack)

    logits = out[:, :, 0:4]
    values = out[:, :, 4:5]
    latents = out[:, :, 5:8]
    eattr = out[:, :, 8:16].reshape(B, N * K)
    src = out[:, :, 16:24].astype(jnp.int32).reshape(B, N * K)
    dst = jnp.broadcast_to(jnp.repeat(jnp.arange(N, dtype=jnp.int32), K)[None, :], (B, N * K))
    eidx = jnp.stack([src, dst], axis=1)
    return (batch[:, :, :4], batch[:, :, 4:5], logits, values, latents, eidx, eattr)


# speedup vs baseline: 110.0496x; 2.0062x over previous
import jax
import jax.numpy as jnp
from jax.experimental import pallas as pl
from jax.experimental.pallas import tpu as pltpu

B, N, IN_DIM, OUT_DIM, HID, K = 64, 256, 5, 3, 64, 8
ALPHA = 0.1
_INTERPRET = False

_W_ENC2, _W_ENC3 = 0, 1
_W_G2L, _W_G2R, _W_G3L, _W_G3R, _W_G4L, _W_G4R = 2, 3, 4, 5, 6, 7

_B_ENC1, _B_ENC2, _B_ENC3 = 0, 1, 2
_B_G = [(3 + 4 * i, 4 + 4 * i, 5 + 4 * i, 6 + 4 * i) for i in range(4)]
_B_LAB, _B_VAL, _B_SKIP = 19, 20, 21
_B_G1WL, _B_G1WR = 22, 23
_B_LABW = 24
_B_VALW = 28
_B_LABB = 29
_B_VALB = 30


def _bf(x):
    return x.astype(jnp.bfloat16)


def _rt(x):
    return x.astype(jnp.bfloat16).astype(jnp.float32)


def _dot(a, b):
    return jnp.dot(a, b, preferred_element_type=jnp.float32)


def _split3(x):
    p0 = _bf(x)
    r1 = x - p0.astype(jnp.float32)
    p1 = _bf(r1)
    p2 = _bf(r1 - p1.astype(jnp.float32))
    return p0, p1, p2


def _mm(w_bf, xT):
    return _dot(w_bf, _bf(xT))


def _gatherT(ohT, xT):
    p0, p1, p2 = _split3(xT)
    return [(_dot(p0, ohT_k) + _dot(p1, ohT_k)) + _dot(p2, ohT_k) for ohT_k in ohT]


def _col(bref, c):
    return bref[:, c:c + 1]


def _attendT(g, xrT, bref, b_idx):
    _, _, att, bias = b_idx
    att_col = _rt(_col(bref, att))
    e = []
    for k in range(K):
        mk = g[k] + xrT
        mk = jnp.where(mk >= 0, mk, 0.2 * mk)
        e.append(jnp.sum(_rt(mk) * att_col, axis=0, keepdims=True))
    e = jnp.concatenate(e, axis=0)
    emax = jnp.max(e, axis=0, keepdims=True)
    ex = jnp.exp(e - emax)
    denom = jnp.sum(ex, axis=0, keepdims=True)
    alpha = ex / denom
    out = _col(bref, bias) + jnp.zeros_like(xrT)
    for k in range(K):
        out = out + alpha[k:k + 1, :] * g[k]
    return out


def _obs_kernel(batch_ref, w1_ref, w_ref, skipw_ref, b_ref, brow_ref, out_ref):
    obs = batch_ref[0]
    h = _dot(_bf(obs), w1_ref[...]) + brow_ref[0:1, :]
    h = jnp.maximum(h, 0.0)
    h = _dot(_bf(h), w_ref[_W_ENC2]) + brow_ref[1:2, :]
    h = jnp.maximum(h, 0.0)
    latent = _dot(_bf(h), w_ref[_W_ENC3]) + brow_ref[2:3, :]

    px = latent[:, 0:1]
    py = latent[:, 1:2]
    x0 = latent[:, 2:3]
    row_i = jax.lax.broadcasted_iota(jnp.int32, (N, N), 0)
    col_i = jax.lax.broadcasted_iota(jnp.int32, (N, N), 1)
    row_f = row_i.astype(jnp.float32)
    diag = row_i == col_i
    xj = jnp.max(jnp.where(diag, px, -3.0e38), axis=0, keepdims=True)
    yj = jnp.max(jnp.where(diag, py, -3.0e38), axis=0, keepdims=True)
    x0_row = jnp.max(jnp.where(diag, x0, -3.0e38), axis=0, keepdims=True)
    dx = px - xj
    dy = py - yj
    d2 = dx * dx + dy * dy + jnp.where(diag, 1e9, 0.0)

    ohT = []
    selT_masks = []
    idx_rows = []
    dist_rows = []
    d2_work = d2
    for k in range(K):
        mk = jnp.min(d2_work, axis=0, keepdims=True)
        cand = jnp.where(d2_work == mk, row_f, 3.0e38)
        idx_f = jnp.min(cand, axis=0, keepdims=True)
        selT = row_f == idx_f
        selT_masks.append(selT)
        ohT.append(selT.astype(jnp.bfloat16))
        idx_rows.append(idx_f)
        dist_rows.append(mk)
        d2_work = jnp.where(selT, 3.0e38, d2_work)
    idx_mat = jnp.concatenate(idx_rows, axis=0)
    dist_mat = jnp.concatenate(dist_rows, axis=0)
    eattrT = jnp.sqrt(jnp.maximum(dist_mat, 1e-12))

    wl1 = _col(b_ref, _B_G1WL)
    bl1 = _col(b_ref, _B_G[0][0])
    br1 = _col(b_ref, _B_G[0][1])
    g1 = []
    for k in range(K):
        gx0 = jnp.sum(jnp.where(selT_masks[k], x0, 0.0), axis=0, keepdims=True)
        g1.append(gx0 * wl1 + bl1)
    xr1T = x0_row * _col(b_ref, _B_G1WR) + br1
    x1T = jnp.maximum(_attendT(g1, xr1T, b_ref, _B_G[0]), 0.0)

    xl2T = _mm(w_ref[_W_G2L], x1T) + _col(b_ref, _B_G[1][0])
    xr2T = _mm(w_ref[_W_G2R], x1T) + _col(b_ref, _B_G[1][1])
    g2 = _gatherT(ohT, xl2T)
    x2T = jnp.maximum(_attendT(g2, xr2T, b_ref, _B_G[1]), 0.0)

    gx2 = _gatherT(ohT, x2T)
    latentT = jnp.transpose(latent)
    skipT = _mm(skipw_ref[...], latentT) + _col(b_ref, _B_SKIP)
    skT = ALPHA * skipT

    def _branch(wl_i, wr_i, b_idx):
        bl = _col(b_ref, b_idx[0])
        g = [_mm(w_ref[wl_i], gx2_k) + bl for gx2_k in gx2]
        xr = _mm(w_ref[wr_i], x2T) + _col(b_ref, b_idx[1])
        return jnp.maximum(_attendT(g, xr, b_ref, b_idx) + skT, 0.0)

    x3T = _branch(_W_G3L, _W_G3R, _B_G[2])
    x4T = _branch(_W_G4L, _W_G4R, _B_G[3])
    lab_rows = [jnp.sum(_rt(x3T) * _rt(_col(b_ref, _B_LABW + c)), axis=0, keepdims=True)
                + b_ref[c:c + 1, _B_LABB:_B_LABB + 1] for c in range(4)]
    logitsT = jnp.concatenate(lab_rows, axis=0)
    valuesT = (jnp.sum(_rt(x4T) * _rt(_col(b_ref, _B_VALW)), axis=0, keepdims=True)
               + b_ref[0:1, _B_VALB:_B_VALB + 1])

    out_tile = jnp.concatenate([
        logitsT,
        valuesT,
        latentT[0:3, :],
        eattrT,
        idx_mat,
        jnp.zeros((8, N), jnp.float32),
    ], axis=0)
    out_ref[0] = out_tile


def _pad_w(w, rows=HID):
    out = jnp.zeros((rows, HID), jnp.float32)
    return out.at[:w.shape[0], :w.shape[1]].set(w)


def _pad_b(b):
    return jnp.zeros((HID,), jnp.float32).at[:b.shape[0]].set(b)


@jax.jit
def kernel(batch, enc_w1, enc_b1, enc_w2, enc_b2, enc_w3, enc_b3,
           g1_wl, g1_bl, g1_wr, g1_br, g1_att, g1_bias,
           g2_wl, g2_bl, g2_wr, g2_br, g2_att, g2_bias,
           g3_wl, g3_bl, g3_wr, g3_br, g3_att, g3_bias,
           g4_wl, g4_bl, g4_wr, g4_br, g4_att, g4_bias,
           label_w, label_b, value_w, value_b, skip_w, skip_b):
    batchp = jnp.pad(batch, ((0, 0), (0, 0), (0, 8 - IN_DIM)))
    w1p = _pad_w(enc_w1, rows=8).astype(jnp.bfloat16)
    wstack = jnp.stack([
        enc_w2,
        _pad_w(enc_w3),
        g2_wl.T, g2_wr.T, g3_wl.T, g3_wr.T, g4_wl.T, g4_wr.T,
    ])
    whi = wstack.astype(jnp.bfloat16)
    skip_wp = _pad_w(skip_w).T.astype(jnp.bfloat16)
    bcols = [enc_b1, enc_b2, _pad_b(enc_b3)]
    for bl, br, att, bias in [(g1_bl, g1_br, g1_att, g1_bias),
                              (g2_bl, g2_br, g2_att, g2_bias),
                              (g3_bl, g3_br, g3_att, g3_bias),
                              (g4_bl, g4_br, g4_att, g4_bias)]:
        bcols += [bl, br, att, bias]
    bcols += [_pad_b(label_b), _pad_b(value_b), skip_b,
              g1_wl[0], g1_wr[0]]
    bcols += [label_w[:, c] for c in range(4)]
    bcols += [value_w[:, 0], _pad_b(label_b), _pad_b(value_b)]
    bstack = jnp.stack(bcols, axis=1)
    bstack = jnp.pad(bstack, ((0, 0), (0, 32 - bstack.shape[1])))
    brow = jnp.stack([enc_b1, enc_b2, _pad_b(enc_b3),
                      jnp.zeros((HID,), jnp.float32), jnp.zeros((HID,), jnp.float32),
                      jnp.zeros((HID,), jnp.float32), jnp.zeros((HID,), jnp.float32),
                      jnp.zeros((HID,), jnp.float32)])

    out = pl.pallas_call(
        _obs_kernel,
        grid=(B,),
        in_specs=[
            pl.BlockSpec((1, N, 8), lambda b: (b, 0, 0)),
            pl.BlockSpec((8, HID), lambda b: (0, 0)),
            pl.BlockSpec((8, HID, HID), lambda b: (0, 0, 0)),
            pl.BlockSpec((HID, HID), lambda b: (0, 0)),
            pl.BlockSpec((HID, 32), lambda b: (0, 0)),
            pl.BlockSpec((8, HID), lambda b: (0, 0)),
        ],
        out_specs=pl.BlockSpec((1, 32, N), lambda b: (b, 0, 0)),
        out_shape=jax.ShapeDtypeStruct((B, 32, N), jnp.float32),
        compiler_params=pltpu.CompilerParams(
            dimension_semantics=("arbitrary",),
        ),
        interpret=_INTERPRET,
    )(batchp, w1p, whi, skip_wp, bstack, brow)

    logits = jnp.swapaxes(out[:, 0:4, :], 1, 2)
    values = jnp.swapaxes(out[:, 4:5, :], 1, 2)
    latents = jnp.swapaxes(out[:, 5:8, :], 1, 2)
    eattr = jnp.swapaxes(out[:, 8:16, :], 1, 2).reshape(B, N * K)
    src = jnp.swapaxes(out[:, 16:24, :], 1, 2).astype(jnp.int32).reshape(B, N * K)
    dst = jnp.broadcast_to(jnp.repeat(jnp.arange(N, dtype=jnp.int32), K)[None, :], (B, N * K))
    eidx = jnp.stack([src, dst], axis=1)
    return (batch[:, :, :4], batch[:, :, 4:5], logits, values, latents, eidx, eattr)


# speedup vs baseline: 125.5017x; 1.1404x over previous
import jax
import jax.numpy as jnp
from jax.experimental import pallas as pl
from jax.experimental.pallas import tpu as pltpu

B, N, IN_DIM, OUT_DIM, HID, K = 64, 256, 5, 3, 64, 8
ALPHA = 0.1
_INTERPRET = False

_W_ENC2, _W_ENC3 = 0, 1
_W_G2L, _W_G2R, _W_G3L, _W_G3R, _W_G4L, _W_G4R = 2, 3, 4, 5, 6, 7

_B_ENC1, _B_ENC2, _B_ENC3 = 0, 1, 2
_B_G = [(3 + 4 * i, 4 + 4 * i, 5 + 4 * i, 6 + 4 * i) for i in range(4)]
_B_LAB, _B_VAL, _B_SKIP = 19, 20, 21
_B_G1WL, _B_G1WR = 22, 23
_B_LABW = 24
_B_VALW = 28
_B_LABB = 29
_B_VALB = 30


def _bf(x):
    return x.astype(jnp.bfloat16)


def _rt(x):
    return x.astype(jnp.bfloat16).astype(jnp.float32)


def _dot(a, b):
    return jnp.dot(a, b, preferred_element_type=jnp.float32)


def _split3(x):
    p0 = _bf(x)
    r1 = x - p0.astype(jnp.float32)
    p1 = _bf(r1)
    p2 = _bf(r1 - p1.astype(jnp.float32))
    return p0, p1, p2


def _mm(w_bf, xT):
    return _dot(w_bf, _bf(xT))


def _gatherT(ohT, xT):
    p = jnp.concatenate(_split3(xT), axis=0)
    out = []
    for ohT_k in ohT:
        t = _dot(p, ohT_k)
        out.append((t[0:HID, :] + t[HID:2 * HID, :]) + t[2 * HID:3 * HID, :])
    return out


def _col(bref, c):
    return bref[:, c:c + 1]


def _attendT(g, xrT, bref, b_idx):
    _, _, att, bias = b_idx
    att_col = _rt(_col(bref, att))
    e = []
    for k in range(K):
        mk = g[k] + xrT
        mk = jnp.where(mk >= 0, mk, 0.2 * mk)
        e.append(jnp.sum(_rt(mk) * att_col, axis=0, keepdims=True))
    e = jnp.concatenate(e, axis=0)
    emax = jnp.max(e, axis=0, keepdims=True)
    ex = jnp.exp(e - emax)
    denom = jnp.sum(ex, axis=0, keepdims=True)
    alpha = ex / denom
    out = _col(bref, bias) + jnp.zeros_like(xrT)
    for k in range(K):
        out = out + alpha[k:k + 1, :] * g[k]
    return out


def _obs_kernel(batch_ref, w1_ref, w_ref, skipw_ref, b_ref, brow_ref, out_ref):
    obs = batch_ref[0]
    h = _dot(_bf(obs), w1_ref[...]) + brow_ref[0:1, :]
    h = jnp.maximum(h, 0.0)
    h = _dot(_bf(h), w_ref[_W_ENC2]) + brow_ref[1:2, :]
    h = jnp.maximum(h, 0.0)
    latent = _dot(_bf(h), w_ref[_W_ENC3]) + brow_ref[2:3, :]

    px = latent[:, 0:1]
    py = latent[:, 1:2]
    x0 = latent[:, 2:3]
    row_i = jax.lax.broadcasted_iota(jnp.int32, (N, N), 0)
    col_i = jax.lax.broadcasted_iota(jnp.int32, (N, N), 1)
    row_f = row_i.astype(jnp.float32)
    diag = row_i == col_i
    xj = jnp.max(jnp.where(diag, px, -3.0e38), axis=0, keepdims=True)
    yj = jnp.max(jnp.where(diag, py, -3.0e38), axis=0, keepdims=True)
    x0_row = jnp.max(jnp.where(diag, x0, -3.0e38), axis=0, keepdims=True)
    dx = px - xj
    dy = py - yj
    d2 = dx * dx + dy * dy + jnp.where(diag, 1e9, 0.0)

    ohT = []
    selT_masks = []
    idx_rows = []
    dist_rows = []
    d2_work = d2
    for k in range(K):
        mk = jnp.min(d2_work, axis=0, keepdims=True)
        cand = jnp.where(d2_work == mk, row_f, 3.0e38)
        idx_f = jnp.min(cand, axis=0, keepdims=True)
        selT = row_f == idx_f
        selT_masks.append(selT)
        ohT.append(selT.astype(jnp.bfloat16))
        idx_rows.append(idx_f)
        dist_rows.append(mk)
        d2_work = jnp.where(selT, 3.0e38, d2_work)
    idx_mat = jnp.concatenate(idx_rows, axis=0)
    dist_mat = jnp.concatenate(dist_rows, axis=0)
    eattrT = jnp.sqrt(jnp.maximum(dist_mat, 1e-12))

    wl1 = _col(b_ref, _B_G1WL)
    bl1 = _col(b_ref, _B_G[0][0])
    br1 = _col(b_ref, _B_G[0][1])
    g1 = []
    for k in range(K):
        gx0 = jnp.sum(jnp.where(selT_masks[k], x0, 0.0), axis=0, keepdims=True)
        g1.append(gx0 * wl1 + bl1)
    xr1T = x0_row * _col(b_ref, _B_G1WR) + br1
    x1T = jnp.maximum(_attendT(g1, xr1T, b_ref, _B_G[0]), 0.0)

    xl2T = _mm(w_ref[_W_G2L], x1T) + _col(b_ref, _B_G[1][0])
    xr2T = _mm(w_ref[_W_G2R], x1T) + _col(b_ref, _B_G[1][1])
    g2 = _gatherT(ohT, xl2T)
    x2T = jnp.maximum(_attendT(g2, xr2T, b_ref, _B_G[1]), 0.0)

    gx2 = _gatherT(ohT, x2T)
    latentT = jnp.transpose(latent)
    skipT = _mm(skipw_ref[...], latentT) + _col(b_ref, _B_SKIP)
    skT = ALPHA * skipT

    def _branch(wl_i, wr_i, b_idx):
        bl = _col(b_ref, b_idx[0])
        g = [_mm(w_ref[wl_i], gx2_k) + bl for gx2_k in gx2]
        xr = _mm(w_ref[wr_i], x2T) + _col(b_ref, b_idx[1])
        return jnp.maximum(_attendT(g, xr, b_ref, b_idx) + skT, 0.0)

    x3T = _branch(_W_G3L, _W_G3R, _B_G[2])
    x4T = _branch(_W_G4L, _W_G4R, _B_G[3])
    lab_rows = [jnp.sum(_rt(x3T) * _rt(_col(b_ref, _B_LABW + c)), axis=0, keepdims=True)
                + b_ref[c:c + 1, _B_LABB:_B_LABB + 1] for c in range(4)]
    logitsT = jnp.concatenate(lab_rows, axis=0)
    valuesT = (jnp.sum(_rt(x4T) * _rt(_col(b_ref, _B_VALW)), axis=0, keepdims=True)
               + b_ref[0:1, _B_VALB:_B_VALB + 1])

    out_tile = jnp.concatenate([
        logitsT,
        valuesT,
        latentT[0:3, :],
        eattrT,
        idx_mat,
        jnp.zeros((8, N), jnp.float32),
    ], axis=0)
    out_ref[0] = out_tile


def _pad_w(w, rows=HID):
    out = jnp.zeros((rows, HID), jnp.float32)
    return out.at[:w.shape[0], :w.shape[1]].set(w)


def _pad_b(b):
    return jnp.zeros((HID,), jnp.float32).at[:b.shape[0]].set(b)


@jax.jit
def kernel(batch, enc_w1, enc_b1, enc_w2, enc_b2, enc_w3, enc_b3,
           g1_wl, g1_bl, g1_wr, g1_br, g1_att, g1_bias,
           g2_wl, g2_bl, g2_wr, g2_br, g2_att, g2_bias,
           g3_wl, g3_bl, g3_wr, g3_br, g3_att, g3_bias,
           g4_wl, g4_bl, g4_wr, g4_br, g4_att, g4_bias,
           label_w, label_b, value_w, value_b, skip_w, skip_b):
    batchp = jnp.pad(batch, ((0, 0), (0, 0), (0, 8 - IN_DIM)))
    w1p = _pad_w(enc_w1, rows=8).astype(jnp.bfloat16)
    wstack = jnp.stack([
        enc_w2,
        _pad_w(enc_w3),
        g2_wl.T, g2_wr.T, g3_wl.T, g3_wr.T, g4_wl.T, g4_wr.T,
    ])
    whi = wstack.astype(jnp.bfloat16)
    skip_wp = _pad_w(skip_w).T.astype(jnp.bfloat16)
    bcols = [enc_b1, enc_b2, _pad_b(enc_b3)]
    for bl, br, att, bias in [(g1_bl, g1_br, g1_att, g1_bias),
                              (g2_bl, g2_br, g2_att, g2_bias),
                              (g3_bl, g3_br, g3_att, g3_bias),
                              (g4_bl, g4_br, g4_att, g4_bias)]:
        bcols += [bl, br, att, bias]
    bcols += [_pad_b(label_b), _pad_b(value_b), skip_b,
              g1_wl[0], g1_wr[0]]
    bcols += [label_w[:, c] for c in range(4)]
    bcols += [value_w[:, 0], _pad_b(label_b), _pad_b(value_b)]
    bstack = jnp.stack(bcols, axis=1)
    bstack = jnp.pad(bstack, ((0, 0), (0, 32 - bstack.shape[1])))
    brow = jnp.stack([enc_b1, enc_b2, _pad_b(enc_b3),
                      jnp.zeros((HID,), jnp.float32), jnp.zeros((HID,), jnp.float32),
                      jnp.zeros((HID,), jnp.float32), jnp.zeros((HID,), jnp.float32),
                      jnp.zeros((HID,), jnp.float32)])

    out = pl.pallas_call(
        _obs_kernel,
        grid=(B,),
        in_specs=[
            pl.BlockSpec((1, N, 8), lambda b: (b, 0, 0)),
            pl.BlockSpec((8, HID), lambda b: (0, 0)),
            pl.BlockSpec((8, HID, HID), lambda b: (0, 0, 0)),
            pl.BlockSpec((HID, HID), lambda b: (0, 0)),
            pl.BlockSpec((HID, 32), lambda b: (0, 0)),
            pl.BlockSpec((8, HID), lambda b: (0, 0)),
        ],
        out_specs=pl.BlockSpec((1, 32, N), lambda b: (b, 0, 0)),
        out_shape=jax.ShapeDtypeStruct((B, 32, N), jnp.float32),
        compiler_params=pltpu.CompilerParams(
            dimension_semantics=("arbitrary",),
        ),
        interpret=_INTERPRET,
    )(batchp, w1p, whi, skip_wp, bstack, brow)

    logits = jnp.swapaxes(out[:, 0:4, :], 1, 2)
    values = jnp.swapaxes(out[:, 4:5, :], 1, 2)
    latents = jnp.swapaxes(out[:, 5:8, :], 1, 2)
    eattr = jnp.swapaxes(out[:, 8:16, :], 1, 2).reshape(B, N * K)
    src = jnp.swapaxes(out[:, 16:24, :], 1, 2).astype(jnp.int32).reshape(B, N * K)
    dst = jnp.broadcast_to(jnp.repeat(jnp.arange(N, dtype=jnp.int32), K)[None, :], (B, N * K))
    eidx = jnp.stack([src, dst], axis=1)
    return (batch[:, :, :4], batch[:, :, 4:5], logits, values, latents, eidx, eattr)


# speedup vs baseline: 126.7789x; 1.0102x over previous
import jax
import jax.numpy as jnp
from jax.experimental import pallas as pl
from jax.experimental.pallas import tpu as pltpu

B, N, IN_DIM, OUT_DIM, HID, K = 64, 256, 5, 3, 64, 8
ALPHA = 0.1
_INTERPRET = False

_W_ENC2, _W_ENC3 = 0, 1
_W_G2L, _W_G2R, _W_G3L, _W_G3R, _W_G4L, _W_G4R = 2, 3, 4, 5, 6, 7

_B_ENC1, _B_ENC2, _B_ENC3 = 0, 1, 2
_B_G = [(3 + 4 * i, 4 + 4 * i, 5 + 4 * i, 6 + 4 * i) for i in range(4)]
_B_LAB, _B_VAL, _B_SKIP = 19, 20, 21
_B_G1WL, _B_G1WR = 22, 23
_B_LABW = 24
_B_VALW = 28
_B_LABB = 29
_B_VALB = 30


def _bf(x):
    return x.astype(jnp.bfloat16)


def _rt(x):
    return x.astype(jnp.bfloat16).astype(jnp.float32)


def _dot(a, b):
    return jnp.dot(a, b, preferred_element_type=jnp.float32)


def _split3(x):
    p0 = _bf(x)
    r1 = x - p0.astype(jnp.float32)
    p1 = _bf(r1)
    p2 = _bf(r1 - p1.astype(jnp.float32))
    return p0, p1, p2


def _mm(w_bf, xT):
    return _dot(w_bf, _bf(xT))


def _gatherT(ohT, xT):
    p = jnp.concatenate(_split3(xT), axis=0)
    out = []
    for ohT_k in ohT:
        t = _dot(p, ohT_k)
        out.append((t[0:HID, :] + t[HID:2 * HID, :]) + t[2 * HID:3 * HID, :])
    return out


def _col(bref, c):
    return bref[:, c:c + 1]


def _attendT(g, xrT, bref, b_idx):
    _, _, att, bias = b_idx
    att_col = _rt(_col(bref, att))
    e = []
    for k in range(K):
        mk = g[k] + xrT
        mk = jnp.where(mk >= 0, mk, 0.2 * mk)
        e.append(jnp.sum(_rt(mk) * att_col, axis=0, keepdims=True))
    e = jnp.concatenate(e, axis=0)
    emax = jnp.max(e, axis=0, keepdims=True)
    ex = jnp.exp(e - emax)
    denom = jnp.sum(ex, axis=0, keepdims=True)
    alpha = ex / denom
    out = _col(bref, bias) + jnp.zeros_like(xrT)
    for k in range(K):
        out = out + alpha[k:k + 1, :] * g[k]
    return out


_OBS_PER_STEP = 2


def _obs_kernel(batch_ref, w1_ref, w_ref, skipw_ref, b_ref, brow_ref, out_ref):
  for _s in range(_OBS_PER_STEP):
    obs = batch_ref[_s]
    h = _dot(_bf(obs), w1_ref[...]) + brow_ref[0:1, :]
    h = jnp.maximum(h, 0.0)
    h = _dot(_bf(h), w_ref[_W_ENC2]) + brow_ref[1:2, :]
    h = jnp.maximum(h, 0.0)
    latent = _dot(_bf(h), w_ref[_W_ENC3]) + brow_ref[2:3, :]

    px = latent[:, 0:1]
    py = latent[:, 1:2]
    x0 = latent[:, 2:3]
    row_i = jax.lax.broadcasted_iota(jnp.int32, (N, N), 0)
    col_i = jax.lax.broadcasted_iota(jnp.int32, (N, N), 1)
    row_f = row_i.astype(jnp.float32)
    diag = row_i == col_i
    xj = jnp.max(jnp.where(diag, px, -3.0e38), axis=0, keepdims=True)
    yj = jnp.max(jnp.where(diag, py, -3.0e38), axis=0, keepdims=True)
    x0_row = jnp.max(jnp.where(diag, x0, -3.0e38), axis=0, keepdims=True)
    dx = px - xj
    dy = py - yj
    d2 = dx * dx + dy * dy + jnp.where(diag, 1e9, 0.0)

    ohT = []
    selT_masks = []
    idx_rows = []
    dist_rows = []
    d2_work = d2
    for k in range(K):
        mk = jnp.min(d2_work, axis=0, keepdims=True)
        cand = jnp.where(d2_work == mk, row_f, 3.0e38)
        idx_f = jnp.min(cand, axis=0, keepdims=True)
        selT = row_f == idx_f
        selT_masks.append(selT)
        ohT.append(selT.astype(jnp.bfloat16))
        idx_rows.append(idx_f)
        dist_rows.append(mk)
        d2_work = jnp.where(selT, 3.0e38, d2_work)
    idx_mat = jnp.concatenate(idx_rows, axis=0)
    dist_mat = jnp.concatenate(dist_rows, axis=0)
    eattrT = jnp.sqrt(jnp.maximum(dist_mat, 1e-12))

    wl1 = _col(b_ref, _B_G1WL)
    bl1 = _col(b_ref, _B_G[0][0])
    br1 = _col(b_ref, _B_G[0][1])
    g1 = []
    for k in range(K):
        gx0 = jnp.sum(jnp.where(selT_masks[k], x0, 0.0), axis=0, keepdims=True)
        g1.append(gx0 * wl1 + bl1)
    xr1T = x0_row * _col(b_ref, _B_G1WR) + br1
    x1T = jnp.maximum(_attendT(g1, xr1T, b_ref, _B_G[0]), 0.0)

    xl2T = _mm(w_ref[_W_G2L], x1T) + _col(b_ref, _B_G[1][0])
    xr2T = _mm(w_ref[_W_G2R], x1T) + _col(b_ref, _B_G[1][1])
    g2 = _gatherT(ohT, xl2T)
    x2T = jnp.maximum(_attendT(g2, xr2T, b_ref, _B_G[1]), 0.0)

    gx2 = _gatherT(ohT, x2T)
    latentT = jnp.transpose(latent)
    skipT = _mm(skipw_ref[...], latentT) + _col(b_ref, _B_SKIP)
    skT = ALPHA * skipT

    def _branch(wl_i, wr_i, b_idx):
        bl = _col(b_ref, b_idx[0])
        g = [_mm(w_ref[wl_i], gx2_k) + bl for gx2_k in gx2]
        xr = _mm(w_ref[wr_i], x2T) + _col(b_ref, b_idx[1])
        return jnp.maximum(_attendT(g, xr, b_ref, b_idx) + skT, 0.0)

    x3T = _branch(_W_G3L, _W_G3R, _B_G[2])
    x4T = _branch(_W_G4L, _W_G4R, _B_G[3])
    lab_rows = [jnp.sum(_rt(x3T) * _rt(_col(b_ref, _B_LABW + c)), axis=0, keepdims=True)
                + b_ref[c:c + 1, _B_LABB:_B_LABB + 1] for c in range(4)]
    logitsT = jnp.concatenate(lab_rows, axis=0)
    valuesT = (jnp.sum(_rt(x4T) * _rt(_col(b_ref, _B_VALW)), axis=0, keepdims=True)
               + b_ref[0:1, _B_VALB:_B_VALB + 1])

    out_tile = jnp.concatenate([
        logitsT,
        valuesT,
        latentT[0:3, :],
        eattrT,
        idx_mat,
        jnp.zeros((8, N), jnp.float32),
    ], axis=0)
    out_ref[_s] = out_tile


def _pad_w(w, rows=HID):
    out = jnp.zeros((rows, HID), jnp.float32)
    return out.at[:w.shape[0], :w.shape[1]].set(w)


def _pad_b(b):
    return jnp.zeros((HID,), jnp.float32).at[:b.shape[0]].set(b)


@jax.jit
def kernel(batch, enc_w1, enc_b1, enc_w2, enc_b2, enc_w3, enc_b3,
           g1_wl, g1_bl, g1_wr, g1_br, g1_att, g1_bias,
           g2_wl, g2_bl, g2_wr, g2_br, g2_att, g2_bias,
           g3_wl, g3_bl, g3_wr, g3_br, g3_att, g3_bias,
           g4_wl, g4_bl, g4_wr, g4_br, g4_att, g4_bias,
           label_w, label_b, value_w, value_b, skip_w, skip_b):
    batchp = jnp.pad(batch, ((0, 0), (0, 0), (0, 8 - IN_DIM)))
    w1p = _pad_w(enc_w1, rows=8).astype(jnp.bfloat16)
    wstack = jnp.stack([
        enc_w2,
        _pad_w(enc_w3),
        g2_wl.T, g2_wr.T, g3_wl.T, g3_wr.T, g4_wl.T, g4_wr.T,
    ])
    whi = wstack.astype(jnp.bfloat16)
    skip_wp = _pad_w(skip_w).T.astype(jnp.bfloat16)
    bcols = [enc_b1, enc_b2, _pad_b(enc_b3)]
    for bl, br, att, bias in [(g1_bl, g1_br, g1_att, g1_bias),
                              (g2_bl, g2_br, g2_att, g2_bias),
                              (g3_bl, g3_br, g3_att, g3_bias),
                              (g4_bl, g4_br, g4_att, g4_bias)]:
        bcols += [bl, br, att, bias]
    bcols += [_pad_b(label_b), _pad_b(value_b), skip_b,
              g1_wl[0], g1_wr[0]]
    bcols += [label_w[:, c] for c in range(4)]
    bcols += [value_w[:, 0], _pad_b(label_b), _pad_b(value_b)]
    bstack = jnp.stack(bcols, axis=1)
    bstack = jnp.pad(bstack, ((0, 0), (0, 32 - bstack.shape[1])))
    brow = jnp.stack([enc_b1, enc_b2, _pad_b(enc_b3),
                      jnp.zeros((HID,), jnp.float32), jnp.zeros((HID,), jnp.float32),
                      jnp.zeros((HID,), jnp.float32), jnp.zeros((HID,), jnp.float32),
                      jnp.zeros((HID,), jnp.float32)])

    out = pl.pallas_call(
        _obs_kernel,
        grid=(B // _OBS_PER_STEP,),
        in_specs=[
            pl.BlockSpec((_OBS_PER_STEP, N, 8), lambda b: (b, 0, 0)),
            pl.BlockSpec((8, HID), lambda b: (0, 0)),
            pl.BlockSpec((8, HID, HID), lambda b: (0, 0, 0)),
            pl.BlockSpec((HID, HID), lambda b: (0, 0)),
            pl.BlockSpec((HID, 32), lambda b: (0, 0)),
            pl.BlockSpec((8, HID), lambda b: (0, 0)),
        ],
        out_specs=pl.BlockSpec((_OBS_PER_STEP, 32, N), lambda b: (b, 0, 0)),
        out_shape=jax.ShapeDtypeStruct((B, 32, N), jnp.float32),
        compiler_params=pltpu.CompilerParams(
            dimension_semantics=("arbitrary",),
        ),
        interpret=_INTERPRET,
    )(batchp, w1p, whi, skip_wp, bstack, brow)

    logits = jnp.swapaxes(out[:, 0:4, :], 1, 2)
    values = jnp.swapaxes(out[:, 4:5, :], 1, 2)
    latents = jnp.swapaxes(out[:, 5:8, :], 1, 2)
    eattr = jnp.swapaxes(out[:, 8:16, :], 1, 2).reshape(B, N * K)
    src = jnp.swapaxes(out[:, 16:24, :], 1, 2).astype(jnp.int32).reshape(B, N * K)
    dst = jnp.broadcast_to(jnp.repeat(jnp.arange(N, dtype=jnp.int32), K)[None, :], (B, N * K))
    eidx = jnp.stack([src, dst], axis=1)
    return (batch[:, :, :4], batch[:, :, 4:5], logits, values, latents, eidx, eattr)
